# Initial kernel scaffold; baseline (speedup 1.0000x reference)
import jax
import jax.numpy as jnp
from jax.experimental import pallas as pl


def kernel(x, edge_index, action_edges, army_counts, params):
    raise NotImplementedError("write your pallas kernel here")


# speedup vs baseline: 4.6751x; 4.6751x over previous
import functools

import jax
import jax.numpy as jnp
from jax import lax
from jax.experimental import pallas as pl
from jax.experimental.pallas import tpu as pltpu
from jax.experimental.pallas import tpu_sc as plsc

NC = 2
NS = 16
NW = NC * NS
C = 80


def _fill2d(ref, nrows, ncols, value):
    nb = ncols // 16
    v16 = jnp.full((16,), value, jnp.float32)

    def row(i, carry):
        for k in range(nb):
            ref[i, pl.ds(k * 16, 16)] = v16
        return carry

    lax.fori_loop(0, nrows, row, 0)


def _make_sage_sc(n_nodes, n_edges, d, with_counts):
    ew = n_edges // NW
    nch = ew // C
    n_pad = ((n_nodes + 8 * NS - 1) // (8 * NS)) * (8 * NS)
    rps = n_pad // NS
    mesh = plsc.VectorSubcoreMesh(
        core_axis_name="c", subcore_axis_name="s",
        num_cores=NC, num_subcores=NS)

    out_type = [jax.ShapeDtypeStruct((NC, n_pad, d), jnp.float32)]
    scratch = [
        pltpu.VMEM((nch, C), jnp.int32),
        pltpu.VMEM((nch, C), jnp.int32),
        pltpu.VMEM((C, d), jnp.float32),
        pltpu.VMEM((rps, d), jnp.float32),
        pltpu.VMEM_SHARED((n_pad, d), jnp.float32),
    ]
    if with_counts:
        out_type.append(jax.ShapeDtypeStruct((NC, n_pad, 16), jnp.float32))
        scratch += [
            pltpu.VMEM((C, 16), jnp.float32),
            pltpu.VMEM((rps, 16), jnp.float32),
            pltpu.VMEM_SHARED((n_pad, 16), jnp.float32),
        ]

    def body(srcR, dstR, h_hbm, agg_out, *rest):
        if with_counts:
            cnt_out, idx_s, idx_d, rows, zrows, agg_sh, ones, zcnt, cnt_sh = rest
        else:
            idx_s, idx_d, rows, zrows, agg_sh = rest
        c = lax.axis_index("c")
        s = lax.axis_index("s")
        wid = s * NC + c

        _fill2d(zrows, rps, d, 0.0)
        pltpu.sync_copy(zrows, agg_sh.at[pl.ds(s * rps, rps)])
        if with_counts:
            _fill2d(ones, C, 16, 1.0)
            _fill2d(zcnt, rps, 16, 0.0)
            pltpu.sync_copy(zcnt, cnt_sh.at[pl.ds(s * rps, rps)])
        plsc.subcore_barrier()

        pltpu.sync_copy(srcR.at[wid], idx_s)
        pltpu.sync_copy(dstR.at[wid], idx_d)

        def chunk(j, carry):
            pltpu.sync_copy(h_hbm.at[idx_s.at[j]], rows)
            pltpu.sync_copy(rows, agg_sh.at[idx_d.at[j]], add=True)
            if with_counts:
                pltpu.sync_copy(ones, cnt_sh.at[idx_d.at[j]], add=True)
            return carry

        lax.fori_loop(0, nch, chunk, 0)
        plsc.subcore_barrier()

        pltpu.sync_copy(agg_sh.at[pl.ds(s * rps, rps)], zrows)
        pltpu.sync_copy(zrows, agg_out.at[c].at[pl.ds(s * rps, rps)])
        if with_counts:
            pltpu.sync_copy(cnt_sh.at[pl.ds(s * rps, rps)], zcnt)
            pltpu.sync_copy(zcnt, cnt_out.at[c].at[pl.ds(s * rps, rps)])

    return pl.kernel(body, out_type=tuple(out_type), mesh=mesh,
                     scratch_types=scratch,
                     compiler_params=pltpu.CompilerParams(
                         use_tc_tiling_on_sc=False))


def _make_edge_gather_sc(n_act, dproj):
    ew = n_act // NW
    nch = ew // C
    mesh = plsc.VectorSubcoreMesh(
        core_axis_name="c", subcore_axis_name="s",
        num_cores=NC, num_subcores=NS)
    out_type = (
        jax.ShapeDtypeStruct((n_act, dproj), jnp.float32),
        jax.ShapeDtypeStruct((n_act, dproj), jnp.float32),
    )
    scratch = [
        pltpu.VMEM((nch, C), jnp.int32),
        pltpu.VMEM((nch, C), jnp.int32),
        pltpu.VMEM((C, dproj), jnp.float32),
        pltpu.VMEM((C, dproj), jnp.float32),
    ]

    def body(srcR, tgtR, gas_hbm, gat_hbm, preS, preT,
             idx_s, idx_t, rows_a, rows_b):
        c = lax.axis_index("c")
        s = lax.axis_index("s")
        wid = s * NC + c
        pltpu.sync_copy(srcR.at[wid], idx_s)
        pltpu.sync_copy(tgtR.at[wid], idx_t)
        base = wid * ew

        def chunk(j, carry):
            pltpu.sync_copy(gas_hbm.at[idx_s.at[j]], rows_a)
            pltpu.sync_copy(rows_a, preS.at[pl.ds(base + j * C, C)])
            pltpu.sync_copy(gat_hbm.at[idx_t.at[j]], rows_b)
            pltpu.sync_copy(rows_b, preT.at[pl.ds(base + j * C, C)])
            return carry

        lax.fori_loop(0, nch, chunk, 0)

    return pl.kernel(body, out_type=out_type, mesh=mesh,
                     scratch_types=scratch,
                     compiler_params=pltpu.CompilerParams(
                         use_tc_tiling_on_sc=False))


def _proj_body(h_ref, wT_ref, out_ref):
    out_ref[...] = jnp.dot(h_ref[...], wT_ref[...],
                           preferred_element_type=jnp.float32)


def _sage_core(n, agg_ref, cnt_ref, h_ref, bl_ref, wrT_ref, g_ref, b_ref):
    a = agg_ref[0][:n] + agg_ref[1][:n]
    cnt = cnt_ref[0][:n, 0:1] + cnt_ref[1][:n, 0:1]
    out = (a / jnp.maximum(cnt, 1.0)
           + jnp.dot(h_ref[...], wrT_ref[...], preferred_element_type=jnp.float32)
           + bl_ref[...])
    nrm = jnp.sqrt(jnp.sum(out * out, axis=1, keepdims=True))
    out = out / jnp.maximum(nrm, 1e-12)
    mu = jnp.mean(out, axis=0, keepdims=True)
    var = jnp.mean((out - mu) ** 2, axis=0, keepdims=True)
    out = (out - mu) / jnp.sqrt(var + 1e-5) * g_ref[...] + b_ref[...]
    return jnp.maximum(out, 0.0)


def _sage_dense_body(n, agg_ref, cnt_ref, h_ref, bl_ref, wrT_ref,
                     g_ref, b_ref, wlnT_ref, out_ref, outp_ref):
    h = _sage_core(n, agg_ref, cnt_ref, h_ref, bl_ref, wrT_ref, g_ref, b_ref)
    out_ref[...] = h
    outp_ref[...] = jnp.dot(h, wlnT_ref[...], preferred_element_type=jnp.float32)


def _final_dense_body(n, agg_ref, cnt_ref, h_ref, bl_ref, wrT_ref,
                      g_ref, b_ref, plw1T_ref, plb1_ref, plg_ref, plb_ref,
                      plw2T_ref, plb2_ref, wsT_ref, wtT_ref,
                      place_ref, gas_ref, gat_ref):
    h3 = _sage_core(n, agg_ref, cnt_ref, h_ref, bl_ref, wrT_ref, g_ref, b_ref)
    hc5 = jnp.clip(h3, -5.0, 5.0)
    t = jnp.dot(hc5, plw1T_ref[...], preferred_element_type=jnp.float32) + plb1_ref[...]
    tmu = jnp.mean(t, axis=0, keepdims=True)
    tvar = jnp.mean((t - tmu) ** 2, axis=0, keepdims=True)
    t = jnp.maximum((t - tmu) / jnp.sqrt(tvar + 1e-5) * plg_ref[...] + plb_ref[...],
                    0.0)
    place = jnp.dot(t, plw2T_ref[...], preferred_element_type=jnp.float32) + plb2_ref[...]
    place_ref[...] = jnp.clip(place, -15.0, 15.0)
    hc3 = jnp.clip(h3, -3.0, 3.0)
    gas_ref[...] = jnp.dot(hc3, wsT_ref[...], preferred_element_type=jnp.float32)
    gat_ref[...] = jnp.dot(hc3, wtT_ref[...], preferred_element_type=jnp.float32)


def _stats_body(preS_ref, preT_ref, o_ref):
    @pl.when(pl.program_id(0) == 0)
    def _():
        o_ref[...] = jnp.zeros_like(o_ref)

    x = preS_ref[...] + preT_ref[...]
    o_ref[0:1, :] += jnp.sum(x, axis=0, keepdims=True)
    o_ref[1:2, :] += jnp.sum(x * x, axis=0, keepdims=True)


def _finalize_body(n_act, preS_ref, preT_ref, stats_ref, esg_ref, esb_ref,
                   esw2T_ref, esb2_ref, asg_ref, asb_ref, asw2T_ref,
                   asb2_ref, attack_ref, army_ref):
    x = preS_ref[...] + preT_ref[...]
    s = stats_ref[...]
    mu = s[0:1, :] * (1.0 / n_act)
    var = s[1:2, :] * (1.0 / n_act) - mu * mu
    inv = 1.0 / jnp.sqrt(var + 1e-5)
    xn = (x - mu) * inv
    e1 = jnp.maximum(xn[:, :64] * esg_ref[...] + esb_ref[...], 0.0)
    a1 = jnp.maximum(xn[:, 64:] * asg_ref[...] + asb_ref[...], 0.0)
    attack = jnp.dot(e1, esw2T_ref[...], preferred_element_type=jnp.float32) + esb2_ref[...]
    attack_ref[...] = jnp.clip(attack, -15.0, 15.0)
    army = jnp.dot(a1, asw2T_ref[...], preferred_element_type=jnp.float32) + asb2_ref[...]
    army_ref[...] = jnp.clip(army, -15.0, 15.0)


def _full(shape):
    return pl.BlockSpec(shape, lambda *_: tuple(0 for _ in shape))


def kernel(x, edge_index, action_edges, army_counts, params):
    n, dfeat = x.shape
    n_edges = edge_index.shape[1]
    n_act = action_edges.shape[0]
    embed = params['sage0_Wl'].shape[0]
    p = params

    src = edge_index[0].astype(jnp.int32).reshape(NW, n_edges // (NW * C), C)
    dst = edge_index[1].astype(jnp.int32).reshape(NW, n_edges // (NW * C), C)
    asrc = action_edges[:, 0].astype(jnp.int32).reshape(NW, n_act // (NW * C), C)
    atgt = action_edges[:, 1].astype(jnp.int32).reshape(NW, n_act // (NW * C), C)

    sage0_sc = _make_sage_sc(n, n_edges, embed, True)
    sage_sc = _make_sage_sc(n, n_edges, embed, False)

    xp = pl.pallas_call(
        _proj_body,
        out_shape=jax.ShapeDtypeStruct((n, embed), jnp.float32),
    )(x, p['sage0_Wl'].T)

    agg0, cnt = sage0_sc(src, dst, xp)

    def dense(aggv, h, i):
        wrT = p['sage%d_Wr' % i].T
        bl = p['sage%d_bl' % i].reshape(1, -1)
        g = p['sage%d_gamma' % i].reshape(1, -1)
        b = p['sage%d_beta' % i].reshape(1, -1)
        wlnT = p['sage%d_Wl' % (i + 1)].T
        return pl.pallas_call(
            functools.partial(_sage_dense_body, n),
            out_shape=(jax.ShapeDtypeStruct((n, embed), jnp.float32),
                       jax.ShapeDtypeStruct((n, embed), jnp.float32)),
        )(aggv, cnt, h, bl, wrT, g, b, wlnT)

    h1, h1p = dense(agg0, x, 0)
    agg1 = sage_sc(src, dst, h1p)[0]
    h2, h2p = dense(agg1, h1, 1)
    agg2 = sage_sc(src, dst, h2p)[0]

    wsT = jnp.concatenate([p['es_W1'][:, :embed].T, p['as_W1'][:, :embed].T],
                          axis=1)
    wtT = jnp.concatenate([p['es_W1'][:, embed:].T, p['as_W1'][:, embed:].T],
                          axis=1)
    place, gas, gat = pl.pallas_call(
        functools.partial(_final_dense_body, n),
        out_shape=(
            jax.ShapeDtypeStruct((n, 1), jnp.float32),
            jax.ShapeDtypeStruct((n, 2 * embed), jnp.float32),
            jax.ShapeDtypeStruct((n, 2 * embed), jnp.float32),
        ),
    )(agg2, cnt, h2, p['sage2_bl'].reshape(1, -1),
      p['sage2_Wr'].T, p['sage2_gamma'].reshape(1, -1),
      p['sage2_beta'].reshape(1, -1),
      p['pl_W1'].T, p['pl_b1'].reshape(1, -1),
      p['pl_gamma'].reshape(1, -1), p['pl_beta'].reshape(1, -1),
      p['pl_W2'].T, p['pl_b2'].reshape(1, -1), wsT, wtT)

    edge_sc = _make_edge_gather_sc(n_act, 2 * embed)
    preS, preT = edge_sc(asrc, atgt, gas, gat)

    bs = 8000
    grid = (n_act // bs,)
    stats = pl.pallas_call(
        _stats_body,
        grid=grid,
        in_specs=[pl.BlockSpec((bs, 2 * embed), lambda i: (i, 0)),
                  pl.BlockSpec((bs, 2 * embed), lambda i: (i, 0))],
        out_specs=pl.BlockSpec((2, 2 * embed), lambda i: (0, 0)),
        out_shape=jax.ShapeDtypeStruct((2, 2 * embed), jnp.float32),
    )(preS, preT)

    max_army = p['as_W2'].shape[0]
    attack, army = pl.pallas_call(
        functools.partial(_finalize_body, n_act),
        grid=grid,
        in_specs=[pl.BlockSpec((bs, 2 * embed), lambda i: (i, 0)),
                  pl.BlockSpec((bs, 2 * embed), lambda i: (i, 0)),
                  _full((2, 2 * embed)),
                  _full((1, embed)), _full((1, embed)),
                  _full((embed, 1)), _full((1, 1)),
                  _full((1, embed)), _full((1, embed)),
                  _full((embed, max_army)), _full((1, max_army))],
        out_specs=[pl.BlockSpec((bs, 1), lambda i: (i, 0)),
                   pl.BlockSpec((bs, max_army), lambda i: (i, 0))],
        out_shape=(
            jax.ShapeDtypeStruct((n_act, 1), jnp.float32),
            jax.ShapeDtypeStruct((n_act, max_army), jnp.float32),
        ),
    )(preS, preT, stats,
      p['es_gamma'].reshape(1, -1), p['es_beta'].reshape(1, -1),
      p['es_W2'].T, p['es_b2'].reshape(1, -1),
      p['as_gamma'].reshape(1, -1), p['as_beta'].reshape(1, -1),
      p['as_W2'].T, p['as_b2'].reshape(1, -1))

    return place.reshape(n), attack.reshape(n_act), army


# speedup vs baseline: 7.7181x; 1.6509x over previous
import functools

import jax
import jax.numpy as jnp
from jax import lax
from jax.experimental import pallas as pl
from jax.experimental.pallas import tpu as pltpu
from jax.experimental.pallas import tpu_sc as plsc

NC = 2
NS = 16
NW = NC * NS
C = 80


def _fill2d(ref, nrows, ncols, value):
    nb = ncols // 16
    v16 = jnp.full((16,), value, jnp.float32)

    def row(i, carry):
        for k in range(nb):
            ref[i, pl.ds(k * 16, 16)] = v16
        return carry

    lax.fori_loop(0, nrows, row, 0)


def _make_counts_sc(n_nodes, n_edges):
    ew = n_edges // NW
    nch = ew // C
    n_pad = ((n_nodes + 8 * NS - 1) // (8 * NS)) * (8 * NS)
    rps = n_pad // NS
    mesh = plsc.VectorSubcoreMesh(
        core_axis_name="c", subcore_axis_name="s",
        num_cores=NC, num_subcores=NS)
    out_type = jax.ShapeDtypeStruct((NC, n_pad, 16), jnp.float32)
    scratch = [
        pltpu.VMEM((nch, C), jnp.int32),
        pltpu.VMEM((C, 16), jnp.float32),
        pltpu.VMEM((rps, 16), jnp.float32),
        pltpu.VMEM_SHARED((n_pad, 16), jnp.float32),
        pltpu.SemaphoreType.DMA,
    ]

    def body(dstR, cnt_out, idx_d, ones, zcnt, cnt_sh, csem):
        c = lax.axis_index("c")
        s = lax.axis_index("s")
        wid = s * NC + c
        _fill2d(ones, C, 16, 1.0)
        _fill2d(zcnt, rps, 16, 0.0)
        pltpu.sync_copy(zcnt, cnt_sh.at[pl.ds(s * rps, rps)])
        plsc.subcore_barrier()
        pltpu.sync_copy(dstR.at[wid], idx_d)

        def fire(j, carry):
            pltpu.async_copy(ones, cnt_sh.at[idx_d.at[j]], csem, add=True)
            return carry

        lax.fori_loop(0, nch, fire, 0)

        def drain(j, carry):
            pltpu.make_async_copy(ones, cnt_sh.at[idx_d.at[0]], csem).wait()
            return carry

        lax.fori_loop(0, nch, drain, 0)
        plsc.subcore_barrier()
        pltpu.sync_copy(cnt_sh.at[pl.ds(s * rps, rps)], zcnt)
        pltpu.sync_copy(zcnt, cnt_out.at[c].at[pl.ds(s * rps, rps)])

    return pl.kernel(body, out_type=out_type, mesh=mesh,
                     scratch_types=scratch,
                     compiler_params=pltpu.CompilerParams(
                         use_tc_tiling_on_sc=False))


def _make_sage_sc(n_nodes, n_edges, d):
    ew = n_edges // NW
    nch = ew // C
    n_pad = ((n_nodes + 8 * NS - 1) // (8 * NS)) * (8 * NS)
    rps = n_pad // NS
    mesh = plsc.VectorSubcoreMesh(
        core_axis_name="c", subcore_axis_name="s",
        num_cores=NC, num_subcores=NS)

    out_type = jax.ShapeDtypeStruct((NC, n_pad, d), jnp.float32)
    scratch = [
        pltpu.VMEM((nch, C), jnp.int32),
        pltpu.VMEM((nch, C), jnp.int32),
        pltpu.VMEM((C, d), jnp.float32),
        pltpu.VMEM((C, d), jnp.float32),
        pltpu.VMEM((rps, d), jnp.float32),
        pltpu.VMEM_SHARED((n_pad, d), jnp.float32),
        pltpu.SemaphoreType.DMA, pltpu.SemaphoreType.DMA,
        pltpu.SemaphoreType.DMA, pltpu.SemaphoreType.DMA,
    ]

    def body(srcR, dstR, h_hbm, agg_out, idx_s, idx_d, rows, rows2, zrows,
             agg_sh, gsem0, gsem1, ssem0, ssem1):
        c = lax.axis_index("c")
        s = lax.axis_index("s")
        wid = s * NC + c

        _fill2d(zrows, rps, d, 0.0)
        pltpu.sync_copy(zrows, agg_sh.at[pl.ds(s * rps, rps)])
        plsc.subcore_barrier()

        pltpu.sync_copy(srcR.at[wid], idx_s)
        pltpu.sync_copy(dstR.at[wid], idx_d)

        bufs = (rows, rows2)
        gsem = (gsem0, gsem1)
        ssem = (ssem0, ssem1)

        pltpu.async_copy(h_hbm.at[idx_s.at[0]], bufs[0], gsem[0])
        pltpu.async_copy(h_hbm.at[idx_s.at[1]], bufs[1], gsem[1])

        def it(jj, carry):
            for pp in (0, 1):
                j = 2 * jj + pp
                pltpu.make_async_copy(h_hbm.at[idx_s.at[0]], bufs[pp],
                                      gsem[pp]).wait()
                pltpu.async_copy(bufs[pp], agg_sh.at[idx_d.at[j]], ssem[pp],
                                 add=True)
                pltpu.make_async_copy(bufs[pp], agg_sh.at[idx_d.at[0]],
                                      ssem[pp]).wait()

                @pl.when(j + 2 < nch)
                def _():
                    pltpu.async_copy(h_hbm.at[idx_s.at[j + 2]], bufs[pp],
                                     gsem[pp])
            return carry

        lax.fori_loop(0, nch // 2, it, 0)
        if nch % 2:
            j = nch - 1
            pltpu.make_async_copy(h_hbm.at[idx_s.at[0]], bufs[0],
                                  gsem[0]).wait()
            pltpu.async_copy(bufs[0], agg_sh.at[idx_d.at[j]], ssem[0],
                             add=True)
            pltpu.make_async_copy(bufs[0], agg_sh.at[idx_d.at[0]],
                                  ssem[0]).wait()
        plsc.subcore_barrier()

        pltpu.sync_copy(agg_sh.at[pl.ds(s * rps, rps)], zrows)
        pltpu.sync_copy(zrows, agg_out.at[c].at[pl.ds(s * rps, rps)])

    return pl.kernel(body, out_type=out_type, mesh=mesh,
                     scratch_types=scratch,
                     compiler_params=pltpu.CompilerParams(
                         use_tc_tiling_on_sc=False))


def _make_edge_gather_sc(n_act, dproj):
    ew = n_act // NW
    nch = ew // C
    nv = dproj // 16
    mesh = plsc.VectorSubcoreMesh(
        core_axis_name="c", subcore_axis_name="s",
        num_cores=NC, num_subcores=NS)
    out_type = (
        jax.ShapeDtypeStruct((n_act, dproj), jnp.float32),
        jax.ShapeDtypeStruct((NW, 2, dproj), jnp.float32),
    )
    scratch = [
        pltpu.VMEM((nch, C), jnp.int32),
        pltpu.VMEM((nch, C), jnp.int32),
        pltpu.VMEM((C, dproj), jnp.float32),
        pltpu.VMEM((C, dproj), jnp.float32),
        pltpu.VMEM((C, dproj), jnp.float32),
        pltpu.VMEM((C, dproj), jnp.float32),
        pltpu.VMEM((C, dproj), jnp.float32),
        pltpu.VMEM((C, dproj), jnp.float32),
        pltpu.VMEM((2, dproj), jnp.float32),
        pltpu.SemaphoreType.DMA, pltpu.SemaphoreType.DMA,
        pltpu.SemaphoreType.DMA, pltpu.SemaphoreType.DMA,
        pltpu.SemaphoreType.DMA, pltpu.SemaphoreType.DMA,
    ]

    def body(srcR, tgtR, gas_hbm, gat_hbm, pre, stats_out,
             idx_s, idx_t, a0, a1, b0, b1, o0, o1, st,
             ga0, ga1, gb0, gb1, w0, w1):
        c = lax.axis_index("c")
        s = lax.axis_index("s")
        wid = s * NC + c
        pltpu.sync_copy(srcR.at[wid], idx_s)
        pltpu.sync_copy(tgtR.at[wid], idx_t)
        base = wid * ew
        abuf, bbuf, obuf = (a0, a1), (b0, b1), (o0, o1)
        gasem, gbsem, wsem = (ga0, ga1), (gb0, gb1), (w0, w1)

        pltpu.async_copy(gas_hbm.at[idx_s.at[0]], abuf[0], gasem[0])
        pltpu.async_copy(gat_hbm.at[idx_t.at[0]], bbuf[0], gbsem[0])
        pltpu.async_copy(gas_hbm.at[idx_s.at[1]], abuf[1], gasem[1])
        pltpu.async_copy(gat_hbm.at[idx_t.at[1]], bbuf[1], gbsem[1])

        zero = jnp.zeros((16,), jnp.float32)
        stats0 = tuple(zero for _ in range(2 * nv))

        def valu(pp, stats):
            a, b, o = abuf[pp], bbuf[pp], obuf[pp]

            def row(i, stats):
                stats = list(stats)
                for k in range(nv):
                    v = a[i, pl.ds(k * 16, 16)] + b[i, pl.ds(k * 16, 16)]
                    o[i, pl.ds(k * 16, 16)] = v
                    stats[k] = stats[k] + v
                    stats[nv + k] = stats[nv + k] + v * v
                return tuple(stats)

            return lax.fori_loop(0, C, row, stats)

        def it(jj, stats):
            for pp in (0, 1):
                j = 2 * jj + pp
                pltpu.make_async_copy(gas_hbm.at[idx_s.at[0]], abuf[pp],
                                      gasem[pp]).wait()
                pltpu.make_async_copy(gat_hbm.at[idx_t.at[0]], bbuf[pp],
                                      gbsem[pp]).wait()

                @pl.when(jj >= 1)
                def _():
                    pltpu.make_async_copy(
                        obuf[pp], pre.at[pl.ds(base, C)], wsem[pp]).wait()

                stats = valu(pp, stats)
                pltpu.async_copy(obuf[pp], pre.at[pl.ds(base + j * C, C)],
                                 wsem[pp])

                @pl.when(j + 2 < nch)
                def _():
                    pltpu.async_copy(gas_hbm.at[idx_s.at[j + 2]], abuf[pp],
                                     gasem[pp])
                    pltpu.async_copy(gat_hbm.at[idx_t.at[j + 2]], bbuf[pp],
                                     gbsem[pp])
            return stats

        stats = lax.fori_loop(0, nch // 2, it, stats0)
        if nch % 2:
            j = nch - 1
            pltpu.make_async_copy(gas_hbm.at[idx_s.at[0]], abuf[0],
                                  gasem[0]).wait()
            pltpu.make_async_copy(gat_hbm.at[idx_t.at[0]], bbuf[0],
                                  gbsem[0]).wait()
            pltpu.make_async_copy(obuf[0], pre.at[pl.ds(base, C)],
                                  wsem[0]).wait()
            stats = valu(0, stats)
            pltpu.async_copy(obuf[0], pre.at[pl.ds(base + j * C, C)], wsem[0])
            pltpu.make_async_copy(obuf[1], pre.at[pl.ds(base, C)],
                                  wsem[1]).wait()
            pltpu.make_async_copy(obuf[0], pre.at[pl.ds(base, C)],
                                  wsem[0]).wait()
        else:
            pltpu.make_async_copy(obuf[0], pre.at[pl.ds(base, C)],
                                  wsem[0]).wait()
            pltpu.make_async_copy(obuf[1], pre.at[pl.ds(base, C)],
                                  wsem[1]).wait()
        for k in range(nv):
            st[0, pl.ds(k * 16, 16)] = stats[k]
            st[1, pl.ds(k * 16, 16)] = stats[nv + k]
        pltpu.sync_copy(st, stats_out.at[wid])

    return pl.kernel(body, out_type=out_type, mesh=mesh,
                     scratch_types=scratch,
                     compiler_params=pltpu.CompilerParams(
                         use_tc_tiling_on_sc=False))


def _proj_body(h_ref, wT_ref, out_ref):
    out_ref[...] = jnp.dot(h_ref[...], wT_ref[...],
                           preferred_element_type=jnp.float32)


def _sage_core(n, agg_ref, cnt_ref, h_ref, bl_ref, wrT_ref, g_ref, b_ref):
    a = agg_ref[0][:n] + agg_ref[1][:n]
    cnt = cnt_ref[0][:n, 0:1] + cnt_ref[1][:n, 0:1]
    out = (a / jnp.maximum(cnt, 1.0)
           + jnp.dot(h_ref[...], wrT_ref[...], preferred_element_type=jnp.float32)
           + bl_ref[...])
    nrm = jnp.sqrt(jnp.sum(out * out, axis=1, keepdims=True))
    out = out / jnp.maximum(nrm, 1e-12)
    mu = jnp.mean(out, axis=0, keepdims=True)
    var = jnp.mean((out - mu) ** 2, axis=0, keepdims=True)
    out = (out - mu) / jnp.sqrt(var + 1e-5) * g_ref[...] + b_ref[...]
    return jnp.maximum(out, 0.0)


def _sage_dense_body(n, agg_ref, cnt_ref, h_ref, bl_ref, wrT_ref,
                     g_ref, b_ref, wlnT_ref, out_ref, outp_ref):
    h = _sage_core(n, agg_ref, cnt_ref, h_ref, bl_ref, wrT_ref, g_ref, b_ref)
    out_ref[...] = h
    outp_ref[...] = jnp.dot(h, wlnT_ref[...], preferred_element_type=jnp.float32)


def _final_dense_body(n, agg_ref, cnt_ref, h_ref, bl_ref, wrT_ref,
                      g_ref, b_ref, plw1T_ref, plb1_ref, plg_ref, plb_ref,
                      plw2T_ref, plb2_ref, wsT_ref, wtT_ref,
                      place_ref, gas_ref, gat_ref):
    h3 = _sage_core(n, agg_ref, cnt_ref, h_ref, bl_ref, wrT_ref, g_ref, b_ref)
    hc5 = jnp.clip(h3, -5.0, 5.0)
    t = jnp.dot(hc5, plw1T_ref[...], preferred_element_type=jnp.float32) + plb1_ref[...]
    tmu = jnp.mean(t, axis=0, keepdims=True)
    tvar = jnp.mean((t - tmu) ** 2, axis=0, keepdims=True)
    t = jnp.maximum((t - tmu) / jnp.sqrt(tvar + 1e-5) * plg_ref[...] + plb_ref[...],
                    0.0)
    place = jnp.dot(t, plw2T_ref[...], preferred_element_type=jnp.float32) + plb2_ref[...]
    place_ref[...] = jnp.clip(place, -15.0, 15.0)
    hc3 = jnp.clip(h3, -3.0, 3.0)
    gas_ref[...] = jnp.dot(hc3, wsT_ref[...], preferred_element_type=jnp.float32)
    gat_ref[...] = jnp.dot(hc3, wtT_ref[...], preferred_element_type=jnp.float32)


def _finalize_body(n_act, pre_ref, stats_ref, esg_ref, esb_ref,
                   esw2T_ref, esb2_ref, asg_ref, asb_ref, asw2T_ref,
                   asb2_ref, attack_ref, army_ref):
    x = pre_ref[...]
    mu = jnp.sum(stats_ref[:, 0, :], axis=0, keepdims=True) * (1.0 / n_act)
    s2 = jnp.sum(stats_ref[:, 1, :], axis=0, keepdims=True)
    var = s2 * (1.0 / n_act) - mu * mu
    inv = 1.0 / jnp.sqrt(var + 1e-5)
    xn = (x - mu) * inv
    e1 = jnp.maximum(xn[:, :64] * esg_ref[...] + esb_ref[...], 0.0)
    a1 = jnp.maximum(xn[:, 64:] * asg_ref[...] + asb_ref[...], 0.0)
    attack = jnp.dot(e1, esw2T_ref[...], preferred_element_type=jnp.float32) + esb2_ref[...]
    attack_ref[...] = jnp.clip(attack, -15.0, 15.0)
    army = jnp.dot(a1, asw2T_ref[...], preferred_element_type=jnp.float32) + asb2_ref[...]
    army_ref[...] = jnp.clip(army, -15.0, 15.0)


def _full(shape):
    return pl.BlockSpec(shape, lambda *_: tuple(0 for _ in shape))


def kernel(x, edge_index, action_edges, army_counts, params):
    n, dfeat = x.shape
    n_edges = edge_index.shape[1]
    n_act = action_edges.shape[0]
    embed = params['sage0_Wl'].shape[0]
    p = params

    src = edge_index[0].astype(jnp.int32).reshape(NW, n_edges // (NW * C), C)
    dst = edge_index[1].astype(jnp.int32).reshape(NW, n_edges // (NW * C), C)
    asrc = action_edges[:, 0].astype(jnp.int32).reshape(NW, n_act // (NW * C), C)
    atgt = action_edges[:, 1].astype(jnp.int32).reshape(NW, n_act // (NW * C), C)

    counts_sc = _make_counts_sc(n, n_edges)
    sage_sc = _make_sage_sc(n, n_edges, embed)

    xp = pl.pallas_call(
        _proj_body,
        out_shape=jax.ShapeDtypeStruct((n, embed), jnp.float32),
    )(x, p['sage0_Wl'].T)

    cnt = counts_sc(dst)
    agg0 = sage_sc(src, dst, xp)

    def dense(aggv, h, i):
        wrT = p['sage%d_Wr' % i].T
        bl = p['sage%d_bl' % i].reshape(1, -1)
        g = p['sage%d_gamma' % i].reshape(1, -1)
        b = p['sage%d_beta' % i].reshape(1, -1)
        wlnT = p['sage%d_Wl' % (i + 1)].T
        return pl.pallas_call(
            functools.partial(_sage_dense_body, n),
            out_shape=(jax.ShapeDtypeStruct((n, embed), jnp.float32),
                       jax.ShapeDtypeStruct((n, embed), jnp.float32)),
        )(aggv, cnt, h, bl, wrT, g, b, wlnT)

    h1, h1p = dense(agg0, x, 0)
    agg1 = sage_sc(src, dst, h1p)
    h2, h2p = dense(agg1, h1, 1)
    agg2 = sage_sc(src, dst, h2p)

    wsT = jnp.concatenate([p['es_W1'][:, :embed].T, p['as_W1'][:, :embed].T],
                          axis=1)
    wtT = jnp.concatenate([p['es_W1'][:, embed:].T, p['as_W1'][:, embed:].T],
                          axis=1)
    place, gas, gat = pl.pallas_call(
        functools.partial(_final_dense_body, n),
        out_shape=(
            jax.ShapeDtypeStruct((n, 1), jnp.float32),
            jax.ShapeDtypeStruct((n, 2 * embed), jnp.float32),
            jax.ShapeDtypeStruct((n, 2 * embed), jnp.float32),
        ),
    )(agg2, cnt, h2, p['sage2_bl'].reshape(1, -1),
      p['sage2_Wr'].T, p['sage2_gamma'].reshape(1, -1),
      p['sage2_beta'].reshape(1, -1),
      p['pl_W1'].T, p['pl_b1'].reshape(1, -1),
      p['pl_gamma'].reshape(1, -1), p['pl_beta'].reshape(1, -1),
      p['pl_W2'].T, p['pl_b2'].reshape(1, -1), wsT, wtT)

    edge_sc = _make_edge_gather_sc(n_act, 2 * embed)
    pre, stats = edge_sc(asrc, atgt, gas, gat)

    bs = 8000
    grid = (n_act // bs,)
    max_army = p['as_W2'].shape[0]
    attack, army = pl.pallas_call(
        functools.partial(_finalize_body, n_act),
        grid=grid,
        in_specs=[pl.BlockSpec((bs, 2 * embed), lambda i: (i, 0)),
                  _full((NW, 2, 2 * embed)),
                  _full((1, embed)), _full((1, embed)),
                  _full((embed, 1)), _full((1, 1)),
                  _full((1, embed)), _full((1, embed)),
                  _full((embed, max_army)), _full((1, max_army))],
        out_specs=[pl.BlockSpec((bs, 1), lambda i: (i, 0)),
                   pl.BlockSpec((bs, max_army), lambda i: (i, 0))],
        out_shape=(
            jax.ShapeDtypeStruct((n_act, 1), jnp.float32),
            jax.ShapeDtypeStruct((n_act, max_army), jnp.float32),
        ),
    )(pre, stats,
      p['es_gamma'].reshape(1, -1), p['es_beta'].reshape(1, -1),
      p['es_W2'].T, p['es_b2'].reshape(1, -1),
      p['as_gamma'].reshape(1, -1), p['as_beta'].reshape(1, -1),
      p['as_W2'].T, p['as_b2'].reshape(1, -1))

    return place.reshape(n), attack.reshape(n_act), army


# speedup vs baseline: 8.6178x; 1.1166x over previous
import functools

import jax
import jax.numpy as jnp
from jax import lax
from jax.experimental import pallas as pl
from jax.experimental.pallas import tpu as pltpu
from jax.experimental.pallas import tpu_sc as plsc

NC = 2
NS = 16
NW = NC * NS
C = 80


def _fill2d(ref, nrows, ncols, value):
    nb = ncols // 16
    v16 = jnp.full((16,), value, jnp.float32)

    def row(i, carry):
        for k in range(nb):
            ref[i, pl.ds(k * 16, 16)] = v16
        return carry

    lax.fori_loop(0, nrows, row, 0)


def _make_counts_sc(n_nodes, n_edges):
    ew = n_edges // NW
    nch = ew // C
    n_pad = ((n_nodes + 8 * NS - 1) // (8 * NS)) * (8 * NS)
    rps = n_pad // NS
    mesh = plsc.VectorSubcoreMesh(
        core_axis_name="c", subcore_axis_name="s",
        num_cores=NC, num_subcores=NS)
    out_type = jax.ShapeDtypeStruct((NC, n_pad, 16), jnp.float32)
    scratch = [
        pltpu.VMEM((nch, C), jnp.int32),
        pltpu.VMEM((C, 16), jnp.float32),
        pltpu.VMEM((rps, 16), jnp.float32),
        pltpu.VMEM_SHARED((n_pad, 16), jnp.float32),
        pltpu.SemaphoreType.DMA,
    ]

    def body(dstR, cnt_out, idx_d, ones, zcnt, cnt_sh, csem):
        c = lax.axis_index("c")
        s = lax.axis_index("s")
        wid = s * NC + c
        _fill2d(ones, C, 16, 1.0)
        _fill2d(zcnt, rps, 16, 0.0)
        pltpu.sync_copy(zcnt, cnt_sh.at[pl.ds(s * rps, rps)])
        plsc.subcore_barrier()
        pltpu.sync_copy(dstR.at[wid], idx_d)

        def fire(j, carry):
            pltpu.async_copy(ones, cnt_sh.at[idx_d.at[j]], csem, add=True)
            return carry

        lax.fori_loop(0, nch, fire, 0)

        def drain(j, carry):
            pltpu.make_async_copy(ones, cnt_sh.at[idx_d.at[0]], csem).wait()
            return carry

        lax.fori_loop(0, nch, drain, 0)
        plsc.subcore_barrier()
        pltpu.sync_copy(cnt_sh.at[pl.ds(s * rps, rps)], zcnt)
        pltpu.sync_copy(zcnt, cnt_out.at[c].at[pl.ds(s * rps, rps)])

    return pl.kernel(body, out_type=out_type, mesh=mesh,
                     scratch_types=scratch,
                     compiler_params=pltpu.CompilerParams(
                         use_tc_tiling_on_sc=False))


def _make_sage_sc(n_nodes, n_edges, d):
    ew = n_edges // NW
    nch = ew // C
    n_pad = ((n_nodes + 8 * NS - 1) // (8 * NS)) * (8 * NS)
    rps = n_pad // NS
    mesh = plsc.VectorSubcoreMesh(
        core_axis_name="c", subcore_axis_name="s",
        num_cores=NC, num_subcores=NS)

    out_type = jax.ShapeDtypeStruct((NC, n_pad, d), jnp.float32)
    scratch = [
        pltpu.VMEM((nch, C), jnp.int32),
        pltpu.VMEM((nch, C), jnp.int32),
        pltpu.VMEM((C, d), jnp.float32),
        pltpu.VMEM((C, d), jnp.float32),
        pltpu.VMEM((rps, d), jnp.float32),
        pltpu.VMEM_SHARED((n_pad, d), jnp.float32),
        pltpu.SemaphoreType.DMA, pltpu.SemaphoreType.DMA,
        pltpu.SemaphoreType.DMA, pltpu.SemaphoreType.DMA,
    ]

    def body(srcR, dstR, h_hbm, agg_out, idx_s, idx_d, rows, rows2, zrows,
             agg_sh, gsem0, gsem1, ssem0, ssem1):
        c = lax.axis_index("c")
        s = lax.axis_index("s")
        wid = s * NC + c

        _fill2d(zrows, rps, d, 0.0)
        pltpu.sync_copy(zrows, agg_sh.at[pl.ds(s * rps, rps)])
        plsc.subcore_barrier()

        pltpu.sync_copy(srcR.at[wid], idx_s)
        pltpu.sync_copy(dstR.at[wid], idx_d)

        bufs = (rows, rows2)
        gsem = (gsem0, gsem1)
        ssem = (ssem0, ssem1)

        pltpu.async_copy(h_hbm.at[idx_s.at[0]], bufs[0], gsem[0])
        pltpu.async_copy(h_hbm.at[idx_s.at[1]], bufs[1], gsem[1])

        def it(jj, carry):
            for pp in (0, 1):
                j = 2 * jj + pp
                pltpu.make_async_copy(h_hbm.at[idx_s.at[0]], bufs[pp],
                                      gsem[pp]).wait()
                pltpu.async_copy(bufs[pp], agg_sh.at[idx_d.at[j]], ssem[pp],
                                 add=True)
                pltpu.make_async_copy(bufs[pp], agg_sh.at[idx_d.at[0]],
                                      ssem[pp]).wait()

                @pl.when(j + 2 < nch)
                def _():
                    pltpu.async_copy(h_hbm.at[idx_s.at[j + 2]], bufs[pp],
                                     gsem[pp])
            return carry

        lax.fori_loop(0, nch // 2, it, 0)
        if nch % 2:
            j = nch - 1
            pltpu.make_async_copy(h_hbm.at[idx_s.at[0]], bufs[0],
                                  gsem[0]).wait()
            pltpu.async_copy(bufs[0], agg_sh.at[idx_d.at[j]], ssem[0],
                             add=True)
            pltpu.make_async_copy(bufs[0], agg_sh.at[idx_d.at[0]],
                                  ssem[0]).wait()
        plsc.subcore_barrier()

        pltpu.sync_copy(agg_sh.at[pl.ds(s * rps, rps)], zrows)
        pltpu.sync_copy(zrows, agg_out.at[c].at[pl.ds(s * rps, rps)])

    return pl.kernel(body, out_type=out_type, mesh=mesh,
                     scratch_types=scratch,
                     compiler_params=pltpu.CompilerParams(
                         use_tc_tiling_on_sc=False))


def _make_edge_gather_sc(n_act, dproj):
    ew = n_act // NW
    nch = ew // C
    nv = dproj // 16
    mesh = plsc.VectorSubcoreMesh(
        core_axis_name="c", subcore_axis_name="s",
        num_cores=NC, num_subcores=NS)
    out_type = (
        jax.ShapeDtypeStruct((n_act, dproj), jnp.float32),
        jax.ShapeDtypeStruct((NW, 2, dproj), jnp.float32),
    )
    scratch = [
        pltpu.VMEM((nch, C), jnp.int32),
        pltpu.VMEM((nch, C), jnp.int32),
        pltpu.VMEM((C, dproj), jnp.float32),
        pltpu.VMEM((C, dproj), jnp.float32),
        pltpu.VMEM((C, dproj), jnp.float32),
        pltpu.VMEM((C, dproj), jnp.float32),
        pltpu.VMEM((C, dproj), jnp.float32),
        pltpu.VMEM((C, dproj), jnp.float32),
        pltpu.VMEM((2, dproj), jnp.float32),
        pltpu.SemaphoreType.DMA, pltpu.SemaphoreType.DMA,
        pltpu.SemaphoreType.DMA, pltpu.SemaphoreType.DMA,
        pltpu.SemaphoreType.DMA, pltpu.SemaphoreType.DMA,
    ]

    def body(srcR, tgtR, gas_hbm, gat_hbm, pre, stats_out,
             idx_s, idx_t, a0, a1, b0, b1, o0, o1, st,
             ga0, ga1, gb0, gb1, w0, w1):
        c = lax.axis_index("c")
        s = lax.axis_index("s")
        wid = s * NC + c
        pltpu.sync_copy(srcR.at[wid], idx_s)
        pltpu.sync_copy(tgtR.at[wid], idx_t)
        base = wid * ew
        abuf, bbuf, obuf = (a0, a1), (b0, b1), (o0, o1)
        gasem, gbsem, wsem = (ga0, ga1), (gb0, gb1), (w0, w1)

        pltpu.async_copy(gas_hbm.at[idx_s.at[0]], abuf[0], gasem[0])
        pltpu.async_copy(gat_hbm.at[idx_t.at[0]], bbuf[0], gbsem[0])
        pltpu.async_copy(gas_hbm.at[idx_s.at[1]], abuf[1], gasem[1])
        pltpu.async_copy(gat_hbm.at[idx_t.at[1]], bbuf[1], gbsem[1])

        zero = jnp.zeros((16,), jnp.float32)
        stats0 = tuple(zero for _ in range(2 * nv))

        def valu(pp, stats):
            a, b, o = abuf[pp], bbuf[pp], obuf[pp]

            def row(i, stats):
                stats = list(stats)
                for k in range(nv):
                    v = a[i, pl.ds(k * 16, 16)] + b[i, pl.ds(k * 16, 16)]
                    o[i, pl.ds(k * 16, 16)] = v
                    stats[k] = stats[k] + v
                    stats[nv + k] = stats[nv + k] + v * v
                return tuple(stats)

            return lax.fori_loop(0, C, row, stats)

        def it(jj, stats):
            for pp in (0, 1):
                j = 2 * jj + pp
                pltpu.make_async_copy(gas_hbm.at[idx_s.at[0]], abuf[pp],
                                      gasem[pp]).wait()
                pltpu.make_async_copy(gat_hbm.at[idx_t.at[0]], bbuf[pp],
                                      gbsem[pp]).wait()

                @pl.when(jj >= 1)
                def _():
                    pltpu.make_async_copy(
                        obuf[pp], pre.at[pl.ds(base, C)], wsem[pp]).wait()

                stats = valu(pp, stats)
                pltpu.async_copy(obuf[pp], pre.at[pl.ds(base + j * C, C)],
                                 wsem[pp])

                @pl.when(j + 2 < nch)
                def _():
                    pltpu.async_copy(gas_hbm.at[idx_s.at[j + 2]], abuf[pp],
                                     gasem[pp])
                    pltpu.async_copy(gat_hbm.at[idx_t.at[j + 2]], bbuf[pp],
                                     gbsem[pp])
            return stats

        stats = lax.fori_loop(0, nch // 2, it, stats0)
        if nch % 2:
            j = nch - 1
            pltpu.make_async_copy(gas_hbm.at[idx_s.at[0]], abuf[0],
                                  gasem[0]).wait()
            pltpu.make_async_copy(gat_hbm.at[idx_t.at[0]], bbuf[0],
                                  gbsem[0]).wait()
            pltpu.make_async_copy(obuf[0], pre.at[pl.ds(base, C)],
                                  wsem[0]).wait()
            stats = valu(0, stats)
            pltpu.async_copy(obuf[0], pre.at[pl.ds(base + j * C, C)], wsem[0])
            pltpu.make_async_copy(obuf[1], pre.at[pl.ds(base, C)],
                                  wsem[1]).wait()
            pltpu.make_async_copy(obuf[0], pre.at[pl.ds(base, C)],
                                  wsem[0]).wait()
        else:
            pltpu.make_async_copy(obuf[0], pre.at[pl.ds(base, C)],
                                  wsem[0]).wait()
            pltpu.make_async_copy(obuf[1], pre.at[pl.ds(base, C)],
                                  wsem[1]).wait()
        for k in range(nv):
            st[0, pl.ds(k * 16, 16)] = stats[k]
            st[1, pl.ds(k * 16, 16)] = stats[nv + k]
        pltpu.sync_copy(st, stats_out.at[wid])

    return pl.kernel(body, out_type=out_type, mesh=mesh,
                     scratch_types=scratch,
                     compiler_params=pltpu.CompilerParams(
                         use_tc_tiling_on_sc=False))


def _proj_body(h_ref, wT_ref, out_ref):
    out_ref[...] = jnp.dot(h_ref[...], wT_ref[...],
                           preferred_element_type=jnp.float32)


def _sage_core(n, agg_ref, cnt_ref, h_ref, bl_ref, wrT_ref, g_ref, b_ref):
    a = agg_ref[0][:n] + agg_ref[1][:n]
    cnt = cnt_ref[0][:n, 0:1] + cnt_ref[1][:n, 0:1]
    out = (a / jnp.maximum(cnt, 1.0)
           + jnp.dot(h_ref[...], wrT_ref[...], preferred_element_type=jnp.float32)
           + bl_ref[...])
    nrm = jnp.sqrt(jnp.sum(out * out, axis=1, keepdims=True))
    out = out / jnp.maximum(nrm, 1e-12)
    mu = jnp.mean(out, axis=0, keepdims=True)
    var = jnp.mean((out - mu) ** 2, axis=0, keepdims=True)
    out = (out - mu) / jnp.sqrt(var + 1e-5) * g_ref[...] + b_ref[...]
    return jnp.maximum(out, 0.0)


def _sage_dense_body(n, agg_ref, cnt_ref, h_ref, bl_ref, wrT_ref,
                     g_ref, b_ref, wlnT_ref, out_ref, outp_ref):
    h = _sage_core(n, agg_ref, cnt_ref, h_ref, bl_ref, wrT_ref, g_ref, b_ref)
    out_ref[...] = h
    outp_ref[...] = jnp.dot(h, wlnT_ref[...], preferred_element_type=jnp.float32)


def _final_dense_body(n, agg_ref, cnt_ref, h_ref, bl_ref, wrT_ref,
                      g_ref, b_ref, plw1T_ref, plb1_ref, plg_ref, plb_ref,
                      plw2T_ref, plb2_ref, wsT_ref, wtT_ref,
                      place_ref, gas_ref, gat_ref):
    h3 = _sage_core(n, agg_ref, cnt_ref, h_ref, bl_ref, wrT_ref, g_ref, b_ref)
    hc5 = jnp.clip(h3, -5.0, 5.0)
    t = jnp.dot(hc5, plw1T_ref[...], preferred_element_type=jnp.float32) + plb1_ref[...]
    tmu = jnp.mean(t, axis=0, keepdims=True)
    tvar = jnp.mean((t - tmu) ** 2, axis=0, keepdims=True)
    t = jnp.maximum((t - tmu) / jnp.sqrt(tvar + 1e-5) * plg_ref[...] + plb_ref[...],
                    0.0)
    place = jnp.dot(t, plw2T_ref[...], preferred_element_type=jnp.float32) + plb2_ref[...]
    place_ref[...] = jnp.clip(place, -15.0, 15.0)
    hc3 = jnp.clip(h3, -3.0, 3.0)
    gas_ref[...] = jnp.dot(hc3, wsT_ref[...], preferred_element_type=jnp.float32)
    gat_ref[...] = jnp.dot(hc3, wtT_ref[...], preferred_element_type=jnp.float32)


def _finalize_body(n_act, pre_ref, stats_ref, esg_ref, esb_ref,
                   esw2T_ref, esb2_ref, asg_ref, asb_ref, asw2T_ref,
                   asb2_ref, attack_ref, army_ref):
    x = pre_ref[...]
    mu = jnp.sum(stats_ref[:, 0, :], axis=0, keepdims=True) * (1.0 / n_act)
    s2 = jnp.sum(stats_ref[:, 1, :], axis=0, keepdims=True)
    var = s2 * (1.0 / n_act) - mu * mu
    inv = 1.0 / jnp.sqrt(var + 1e-5)
    xn = (x - mu) * inv
    e1 = jnp.maximum(xn[:, :64] * esg_ref[...] + esb_ref[...], 0.0)
    a1 = jnp.maximum(xn[:, 64:] * asg_ref[...] + asb_ref[...], 0.0)
    attack = jnp.dot(e1, esw2T_ref[...], preferred_element_type=jnp.float32) + esb2_ref[...]
    attack_ref[...] = jnp.clip(attack, -15.0, 15.0).reshape(attack_ref.shape)
    army = jnp.dot(a1, asw2T_ref[...], preferred_element_type=jnp.float32) + asb2_ref[...]
    army_ref[...] = jnp.clip(army, -15.0, 15.0)


def _full(shape):
    return pl.BlockSpec(shape, lambda *_: tuple(0 for _ in shape))


def kernel(x, edge_index, action_edges, army_counts, params):
    n, dfeat = x.shape
    n_edges = edge_index.shape[1]
    n_act = action_edges.shape[0]
    embed = params['sage0_Wl'].shape[0]
    p = params

    src = edge_index[0].astype(jnp.int32).reshape(NW, n_edges // (NW * C), C)
    dst = edge_index[1].astype(jnp.int32).reshape(NW, n_edges // (NW * C), C)
    asrc = action_edges[:, 0].astype(jnp.int32).reshape(NW, n_act // (NW * C), C)
    atgt = action_edges[:, 1].astype(jnp.int32).reshape(NW, n_act // (NW * C), C)

    counts_sc = _make_counts_sc(n, n_edges)
    sage_sc = _make_sage_sc(n, n_edges, embed)

    xp = pl.pallas_call(
        _proj_body,
        out_shape=jax.ShapeDtypeStruct((n, embed), jnp.float32),
    )(x, p['sage0_Wl'].T)

    cnt = counts_sc(dst)
    agg0 = sage_sc(src, dst, xp)

    def dense(aggv, h, i):
        wrT = p['sage%d_Wr' % i].T
        bl = p['sage%d_bl' % i].reshape(1, -1)
        g = p['sage%d_gamma' % i].reshape(1, -1)
        b = p['sage%d_beta' % i].reshape(1, -1)
        wlnT = p['sage%d_Wl' % (i + 1)].T
        return pl.pallas_call(
            functools.partial(_sage_dense_body, n),
            out_shape=(jax.ShapeDtypeStruct((n, embed), jnp.float32),
                       jax.ShapeDtypeStruct((n, embed), jnp.float32)),
        )(aggv, cnt, h, bl, wrT, g, b, wlnT)

    h1, h1p = dense(agg0, x, 0)
    agg1 = sage_sc(src, dst, h1p)
    h2, h2p = dense(agg1, h1, 1)
    agg2 = sage_sc(src, dst, h2p)

    wsT = jnp.concatenate([p['es_W1'][:, :embed].T, p['as_W1'][:, :embed].T],
                          axis=1)
    wtT = jnp.concatenate([p['es_W1'][:, embed:].T, p['as_W1'][:, embed:].T],
                          axis=1)
    place, gas, gat = pl.pallas_call(
        functools.partial(_final_dense_body, n),
        out_shape=(
            jax.ShapeDtypeStruct((n, 1), jnp.float32),
            jax.ShapeDtypeStruct((n, 2 * embed), jnp.float32),
            jax.ShapeDtypeStruct((n, 2 * embed), jnp.float32),
        ),
    )(agg2, cnt, h2, p['sage2_bl'].reshape(1, -1),
      p['sage2_Wr'].T, p['sage2_gamma'].reshape(1, -1),
      p['sage2_beta'].reshape(1, -1),
      p['pl_W1'].T, p['pl_b1'].reshape(1, -1),
      p['pl_gamma'].reshape(1, -1), p['pl_beta'].reshape(1, -1),
      p['pl_W2'].T, p['pl_b2'].reshape(1, -1), wsT, wtT)

    edge_sc = _make_edge_gather_sc(n_act, 2 * embed)
    pre, stats = edge_sc(asrc, atgt, gas, gat)

    bs = 16384
    grid = ((n_act + bs - 1) // bs,)
    max_army = p['as_W2'].shape[0]
    arows = bs * max_army // 128
    attack2d, army2d = pl.pallas_call(
        functools.partial(_finalize_body, n_act),
        grid=grid,
        in_specs=[pl.BlockSpec((bs, 2 * embed), lambda i: (i, 0)),
                  _full((NW, 2, 2 * embed)),
                  _full((1, embed)), _full((1, embed)),
                  _full((embed, 1)), _full((1, 1)),
                  _full((1, embed)), _full((1, embed)),
                  _full((embed, max_army)), _full((1, max_army))],
        out_specs=[pl.BlockSpec((bs // 128, 128), lambda i: (i, 0)),
                   pl.BlockSpec((bs, max_army), lambda i: (i, 0))],
        out_shape=(
            jax.ShapeDtypeStruct((n_act // 128, 128), jnp.float32),
            jax.ShapeDtypeStruct((n_act, max_army), jnp.float32),
        ),
    )(pre, stats,
      p['es_gamma'].reshape(1, -1), p['es_beta'].reshape(1, -1),
      p['es_W2'].T, p['es_b2'].reshape(1, -1),
      p['as_gamma'].reshape(1, -1), p['as_beta'].reshape(1, -1),
      p['as_W2'].T, p['as_b2'].reshape(1, -1))

    return place.reshape(n), attack2d.reshape(n_act), army2d


# speedup vs baseline: 9.6094x; 1.1151x over previous
import functools

import jax
import jax.numpy as jnp
from jax import lax
from jax.experimental import pallas as pl
from jax.experimental.pallas import tpu as pltpu
from jax.experimental.pallas import tpu_sc as plsc

NC = 2
NS = 16
NW = NC * NS
C = 80


def _fill2d(ref, nrows, ncols, value):
    nb = ncols // 16
    v16 = jnp.full((16,), value, jnp.float32)

    def row(i, carry):
        for k in range(nb):
            ref[i, pl.ds(k * 16, 16)] = v16
        return carry

    lax.fori_loop(0, nrows, row, 0)


def _make_counts_sc(n_nodes, n_edges):
    ew = n_edges // NW
    nch = ew // C
    n_pad = ((n_nodes + 8 * NS - 1) // (8 * NS)) * (8 * NS)
    rps = n_pad // NS
    mesh = plsc.VectorSubcoreMesh(
        core_axis_name="c", subcore_axis_name="s",
        num_cores=NC, num_subcores=NS)
    out_type = jax.ShapeDtypeStruct((NC, n_pad, 16), jnp.float32)
    scratch = [
        pltpu.VMEM((nch, C), jnp.int32),
        pltpu.VMEM((C, 16), jnp.float32),
        pltpu.VMEM((rps, 16), jnp.float32),
        pltpu.VMEM_SHARED((n_pad, 16), jnp.float32),
        pltpu.SemaphoreType.DMA,
    ]

    def body(dstR, cnt_out, idx_d, ones, zcnt, cnt_sh, csem):
        c = lax.axis_index("c")
        s = lax.axis_index("s")
        wid = s * NC + c
        _fill2d(ones, C, 16, 1.0)
        _fill2d(zcnt, rps, 16, 0.0)
        pltpu.sync_copy(zcnt, cnt_sh.at[pl.ds(s * rps, rps)])
        plsc.subcore_barrier()
        pltpu.sync_copy(dstR.at[wid], idx_d)

        def fire(j, carry):
            pltpu.async_copy(ones, cnt_sh.at[idx_d.at[j]], csem, add=True)
            return carry

        lax.fori_loop(0, nch, fire, 0)

        def drain(j, carry):
            pltpu.make_async_copy(ones, cnt_sh.at[idx_d.at[0]], csem).wait()
            return carry

        lax.fori_loop(0, nch, drain, 0)
        plsc.subcore_barrier()
        pltpu.sync_copy(cnt_sh.at[pl.ds(s * rps, rps)], zcnt)
        pltpu.sync_copy(zcnt, cnt_out.at[c].at[pl.ds(s * rps, rps)])

    return pl.kernel(body, out_type=out_type, mesh=mesh,
                     scratch_types=scratch,
                     compiler_params=pltpu.CompilerParams(
                         use_tc_tiling_on_sc=False))


def _make_sage_sc(n_nodes, n_edges, d):
    ew = n_edges // NW
    nch = ew // C
    n_pad = ((n_nodes + 8 * NS - 1) // (8 * NS)) * (8 * NS)
    rps = n_pad // NS
    mesh = plsc.VectorSubcoreMesh(
        core_axis_name="c", subcore_axis_name="s",
        num_cores=NC, num_subcores=NS)

    NB = 4
    out_type = jax.ShapeDtypeStruct((NC, n_pad, d), jnp.float32)
    scratch = [
        pltpu.VMEM((nch, C), jnp.int32),
        pltpu.VMEM((nch, C), jnp.int32),
    ] + [pltpu.VMEM((C, d), jnp.float32) for _ in range(NB)] + [
        pltpu.VMEM((rps, d), jnp.float32),
        pltpu.VMEM_SHARED((n_pad, d), jnp.float32),
    ] + [pltpu.SemaphoreType.DMA for _ in range(2 * NB)]

    def body(srcR, dstR, h_hbm, agg_out, idx_s, idx_d, *rest):
        bufs = rest[:NB]
        zrows = rest[NB]
        agg_sh = rest[NB + 1]
        gsem = rest[NB + 2:2 * NB + 2]
        ssem = rest[2 * NB + 2:]
        c = lax.axis_index("c")
        s = lax.axis_index("s")
        wid = s * NC + c

        _fill2d(zrows, rps, d, 0.0)
        pltpu.sync_copy(zrows, agg_sh.at[pl.ds(s * rps, rps)])
        plsc.subcore_barrier()

        pltpu.sync_copy(srcR.at[wid], idx_s)
        pltpu.sync_copy(dstR.at[wid], idx_d)

        for k in range(NB):
            pltpu.async_copy(h_hbm.at[idx_s.at[k]], bufs[k], gsem[k])

        def it(jj, carry):
            for pp in range(NB):
                j = NB * jj + pp
                pltpu.make_async_copy(h_hbm.at[idx_s.at[0]], bufs[pp],
                                      gsem[pp]).wait()
                pltpu.async_copy(bufs[pp], agg_sh.at[idx_d.at[j]], ssem[pp],
                                 add=True)
                pltpu.make_async_copy(bufs[pp], agg_sh.at[idx_d.at[0]],
                                      ssem[pp]).wait()

                @pl.when(j + NB < nch)
                def _():
                    pltpu.async_copy(h_hbm.at[idx_s.at[j + NB]], bufs[pp],
                                     gsem[pp])
            return carry

        lax.fori_loop(0, nch // NB, it, 0)
        for pp in range(nch % NB):
            j = (nch // NB) * NB + pp
            pltpu.make_async_copy(h_hbm.at[idx_s.at[0]], bufs[pp],
                                  gsem[pp]).wait()
            pltpu.async_copy(bufs[pp], agg_sh.at[idx_d.at[j]], ssem[pp],
                             add=True)
            pltpu.make_async_copy(bufs[pp], agg_sh.at[idx_d.at[0]],
                                  ssem[pp]).wait()
        plsc.subcore_barrier()

        pltpu.sync_copy(agg_sh.at[pl.ds(s * rps, rps)], zrows)
        pltpu.sync_copy(zrows, agg_out.at[c].at[pl.ds(s * rps, rps)])

    return pl.kernel(body, out_type=out_type, mesh=mesh,
                     scratch_types=scratch,
                     compiler_params=pltpu.CompilerParams(
                         use_tc_tiling_on_sc=False))


def _make_edge_gather_sc(n_act, dproj):
    ew = n_act // NW
    nch = ew // C
    nv = dproj // 16
    mesh = plsc.VectorSubcoreMesh(
        core_axis_name="c", subcore_axis_name="s",
        num_cores=NC, num_subcores=NS)
    out_type = (
        jax.ShapeDtypeStruct((n_act, dproj), jnp.float32),
        jax.ShapeDtypeStruct((NW, 2, dproj), jnp.float32),
    )
    scratch = [
        pltpu.VMEM((nch, C), jnp.int32),
        pltpu.VMEM((nch, C), jnp.int32),
        pltpu.VMEM((C, dproj), jnp.float32),
        pltpu.VMEM((C, dproj), jnp.float32),
        pltpu.VMEM((C, dproj), jnp.float32),
        pltpu.VMEM((C, dproj), jnp.float32),
        pltpu.VMEM((C, dproj), jnp.float32),
        pltpu.VMEM((C, dproj), jnp.float32),
        pltpu.VMEM((2, dproj), jnp.float32),
        pltpu.SemaphoreType.DMA, pltpu.SemaphoreType.DMA,
        pltpu.SemaphoreType.DMA, pltpu.SemaphoreType.DMA,
        pltpu.SemaphoreType.DMA, pltpu.SemaphoreType.DMA,
    ]

    def body(srcR, tgtR, gas_hbm, gat_hbm, pre, stats_out,
             idx_s, idx_t, a0, a1, b0, b1, o0, o1, st,
             ga0, ga1, gb0, gb1, w0, w1):
        c = lax.axis_index("c")
        s = lax.axis_index("s")
        wid = s * NC + c
        pltpu.sync_copy(srcR.at[wid], idx_s)
        pltpu.sync_copy(tgtR.at[wid], idx_t)
        base = wid * ew
        abuf, bbuf, obuf = (a0, a1), (b0, b1), (o0, o1)
        gasem, gbsem, wsem = (ga0, ga1), (gb0, gb1), (w0, w1)

        pltpu.async_copy(gas_hbm.at[idx_s.at[0]], abuf[0], gasem[0])
        pltpu.async_copy(gat_hbm.at[idx_t.at[0]], bbuf[0], gbsem[0])
        pltpu.async_copy(gas_hbm.at[idx_s.at[1]], abuf[1], gasem[1])
        pltpu.async_copy(gat_hbm.at[idx_t.at[1]], bbuf[1], gbsem[1])

        zero = jnp.zeros((16,), jnp.float32)
        stats0 = tuple(zero for _ in range(2 * nv))

        def valu(pp, stats):
            a, b, o = abuf[pp], bbuf[pp], obuf[pp]

            def row(i, stats):
                stats = list(stats)
                for k in range(nv):
                    v = a[i, pl.ds(k * 16, 16)] + b[i, pl.ds(k * 16, 16)]
                    o[i, pl.ds(k * 16, 16)] = v
                    stats[k] = stats[k] + v
                    stats[nv + k] = stats[nv + k] + v * v
                return tuple(stats)

            return lax.fori_loop(0, C, row, stats)

        def it(jj, stats):
            for pp in (0, 1):
                j = 2 * jj + pp
                pltpu.make_async_copy(gas_hbm.at[idx_s.at[0]], abuf[pp],
                                      gasem[pp]).wait()
                pltpu.make_async_copy(gat_hbm.at[idx_t.at[0]], bbuf[pp],
                                      gbsem[pp]).wait()

                @pl.when(jj >= 1)
                def _():
                    pltpu.make_async_copy(
                        obuf[pp], pre.at[pl.ds(base, C)], wsem[pp]).wait()

                stats = valu(pp, stats)
                pltpu.async_copy(obuf[pp], pre.at[pl.ds(base + j * C, C)],
                                 wsem[pp])

                @pl.when(j + 2 < nch)
                def _():
                    pltpu.async_copy(gas_hbm.at[idx_s.at[j + 2]], abuf[pp],
                                     gasem[pp])
                    pltpu.async_copy(gat_hbm.at[idx_t.at[j + 2]], bbuf[pp],
                                     gbsem[pp])
            return stats

        stats = lax.fori_loop(0, nch // 2, it, stats0)
        if nch % 2:
            j = nch - 1
            pltpu.make_async_copy(gas_hbm.at[idx_s.at[0]], abuf[0],
                                  gasem[0]).wait()
            pltpu.make_async_copy(gat_hbm.at[idx_t.at[0]], bbuf[0],
                                  gbsem[0]).wait()
            pltpu.make_async_copy(obuf[0], pre.at[pl.ds(base, C)],
                                  wsem[0]).wait()
            stats = valu(0, stats)
            pltpu.async_copy(obuf[0], pre.at[pl.ds(base + j * C, C)], wsem[0])
            pltpu.make_async_copy(obuf[1], pre.at[pl.ds(base, C)],
                                  wsem[1]).wait()
            pltpu.make_async_copy(obuf[0], pre.at[pl.ds(base, C)],
                                  wsem[0]).wait()
        else:
            pltpu.make_async_copy(obuf[0], pre.at[pl.ds(base, C)],
                                  wsem[0]).wait()
            pltpu.make_async_copy(obuf[1], pre.at[pl.ds(base, C)],
                                  wsem[1]).wait()
        for k in range(nv):
            st[0, pl.ds(k * 16, 16)] = stats[k]
            st[1, pl.ds(k * 16, 16)] = stats[nv + k]
        pltpu.sync_copy(st, stats_out.at[wid])

    return pl.kernel(body, out_type=out_type, mesh=mesh,
                     scratch_types=scratch,
                     compiler_params=pltpu.CompilerParams(
                         use_tc_tiling_on_sc=False))


def _proj_body(h_ref, wT_ref, out_ref):
    out_ref[...] = jnp.dot(h_ref[...], wT_ref[...],
                           preferred_element_type=jnp.float32)


def _sage_core(n, agg_ref, cnt_ref, h_ref, bl_ref, wrT_ref, g_ref, b_ref):
    a = agg_ref[0][:n] + agg_ref[1][:n]
    cnt = cnt_ref[0][:n, 0:1] + cnt_ref[1][:n, 0:1]
    out = (a / jnp.maximum(cnt, 1.0)
           + jnp.dot(h_ref[...], wrT_ref[...], preferred_element_type=jnp.float32)
           + bl_ref[...])
    nrm = jnp.sqrt(jnp.sum(out * out, axis=1, keepdims=True))
    out = out / jnp.maximum(nrm, 1e-12)
    mu = jnp.mean(out, axis=0, keepdims=True)
    var = jnp.mean((out - mu) ** 2, axis=0, keepdims=True)
    out = (out - mu) / jnp.sqrt(var + 1e-5) * g_ref[...] + b_ref[...]
    return jnp.maximum(out, 0.0)


def _sage_dense_body(n, agg_ref, cnt_ref, h_ref, bl_ref, wrT_ref,
                     g_ref, b_ref, wlnT_ref, out_ref, outp_ref):
    h = _sage_core(n, agg_ref, cnt_ref, h_ref, bl_ref, wrT_ref, g_ref, b_ref)
    out_ref[...] = h
    outp_ref[...] = jnp.dot(h, wlnT_ref[...], preferred_element_type=jnp.float32)


def _final_dense_body(n, agg_ref, cnt_ref, h_ref, bl_ref, wrT_ref,
                      g_ref, b_ref, plw1T_ref, plb1_ref, plg_ref, plb_ref,
                      plw2T_ref, plb2_ref, wsT_ref, wtT_ref,
                      place_ref, gas_ref, gat_ref):
    h3 = _sage_core(n, agg_ref, cnt_ref, h_ref, bl_ref, wrT_ref, g_ref, b_ref)
    hc5 = jnp.clip(h3, -5.0, 5.0)
    t = jnp.dot(hc5, plw1T_ref[...], preferred_element_type=jnp.float32) + plb1_ref[...]
    tmu = jnp.mean(t, axis=0, keepdims=True)
    tvar = jnp.mean((t - tmu) ** 2, axis=0, keepdims=True)
    t = jnp.maximum((t - tmu) / jnp.sqrt(tvar + 1e-5) * plg_ref[...] + plb_ref[...],
                    0.0)
    place = jnp.dot(t, plw2T_ref[...], preferred_element_type=jnp.float32) + plb2_ref[...]
    place_ref[...] = jnp.clip(place, -15.0, 15.0)
    hc3 = jnp.clip(h3, -3.0, 3.0)
    gas_ref[...] = jnp.dot(hc3, wsT_ref[...], preferred_element_type=jnp.float32)
    gat_ref[...] = jnp.dot(hc3, wtT_ref[...], preferred_element_type=jnp.float32)


def _finalize_body(n_act, pre_ref, stats_ref, esg_ref, esb_ref,
                   esw2T_ref, esb2_ref, asg_ref, asb_ref, asw2T_ref,
                   asb2_ref, attack_ref, army_ref):
    x = pre_ref[...]
    mu = jnp.sum(stats_ref[:, 0, :], axis=0, keepdims=True) * (1.0 / n_act)
    s2 = jnp.sum(stats_ref[:, 1, :], axis=0, keepdims=True)
    var = s2 * (1.0 / n_act) - mu * mu
    inv = 1.0 / jnp.sqrt(var + 1e-5)
    xn = (x - mu) * inv
    e1 = jnp.maximum(xn[:, :64] * esg_ref[...] + esb_ref[...], 0.0)
    a1 = jnp.maximum(xn[:, 64:] * asg_ref[...] + asb_ref[...], 0.0)
    attack = jnp.dot(e1, esw2T_ref[...], preferred_element_type=jnp.float32) + esb2_ref[...]
    attack_ref[...] = jnp.clip(attack, -15.0, 15.0).reshape(attack_ref.shape)
    army = jnp.dot(a1, asw2T_ref[...], preferred_element_type=jnp.float32) + asb2_ref[...]
    army_ref[...] = jnp.clip(army, -15.0, 15.0)


def _full(shape):
    return pl.BlockSpec(shape, lambda *_: tuple(0 for _ in shape))


def kernel(x, edge_index, action_edges, army_counts, params):
    n, dfeat = x.shape
    n_edges = edge_index.shape[1]
    n_act = action_edges.shape[0]
    embed = params['sage0_Wl'].shape[0]
    p = params

    src = edge_index[0].astype(jnp.int32).reshape(NW, n_edges // (NW * C), C)
    dst = edge_index[1].astype(jnp.int32).reshape(NW, n_edges // (NW * C), C)
    asrc = action_edges[:, 0].astype(jnp.int32).reshape(NW, n_act // (NW * C), C)
    atgt = action_edges[:, 1].astype(jnp.int32).reshape(NW, n_act // (NW * C), C)

    counts_sc = _make_counts_sc(n, n_edges)
    sage_sc = _make_sage_sc(n, n_edges, embed)

    xp = pl.pallas_call(
        _proj_body,
        out_shape=jax.ShapeDtypeStruct((n, embed), jnp.float32),
    )(x, p['sage0_Wl'].T)

    cnt = counts_sc(dst)
    agg0 = sage_sc(src, dst, xp)

    def dense(aggv, h, i):
        wrT = p['sage%d_Wr' % i].T
        bl = p['sage%d_bl' % i].reshape(1, -1)
        g = p['sage%d_gamma' % i].reshape(1, -1)
        b = p['sage%d_beta' % i].reshape(1, -1)
        wlnT = p['sage%d_Wl' % (i + 1)].T
        return pl.pallas_call(
            functools.partial(_sage_dense_body, n),
            out_shape=(jax.ShapeDtypeStruct((n, embed), jnp.float32),
                       jax.ShapeDtypeStruct((n, embed), jnp.float32)),
        )(aggv, cnt, h, bl, wrT, g, b, wlnT)

    h1, h1p = dense(agg0, x, 0)
    agg1 = sage_sc(src, dst, h1p)
    h2, h2p = dense(agg1, h1, 1)
    agg2 = sage_sc(src, dst, h2p)

    wsT = jnp.concatenate([p['es_W1'][:, :embed].T, p['as_W1'][:, :embed].T],
                          axis=1)
    wtT = jnp.concatenate([p['es_W1'][:, embed:].T, p['as_W1'][:, embed:].T],
                          axis=1)
    place, gas, gat = pl.pallas_call(
        functools.partial(_final_dense_body, n),
        out_shape=(
            jax.ShapeDtypeStruct((n, 1), jnp.float32),
            jax.ShapeDtypeStruct((n, 2 * embed), jnp.float32),
            jax.ShapeDtypeStruct((n, 2 * embed), jnp.float32),
        ),
    )(agg2, cnt, h2, p['sage2_bl'].reshape(1, -1),
      p['sage2_Wr'].T, p['sage2_gamma'].reshape(1, -1),
      p['sage2_beta'].reshape(1, -1),
      p['pl_W1'].T, p['pl_b1'].reshape(1, -1),
      p['pl_gamma'].reshape(1, -1), p['pl_beta'].reshape(1, -1),
      p['pl_W2'].T, p['pl_b2'].reshape(1, -1), wsT, wtT)

    edge_sc = _make_edge_gather_sc(n_act, 2 * embed)
    pre, stats = edge_sc(asrc, atgt, gas, gat)

    bs = 16384
    grid = ((n_act + bs - 1) // bs,)
    max_army = p['as_W2'].shape[0]
    arows = bs * max_army // 128
    attack2d, army2d = pl.pallas_call(
        functools.partial(_finalize_body, n_act),
        grid=grid,
        in_specs=[pl.BlockSpec((bs, 2 * embed), lambda i: (i, 0)),
                  _full((NW, 2, 2 * embed)),
                  _full((1, embed)), _full((1, embed)),
                  _full((embed, 1)), _full((1, 1)),
                  _full((1, embed)), _full((1, embed)),
                  _full((embed, max_army)), _full((1, max_army))],
        out_specs=[pl.BlockSpec((bs // 128, 128), lambda i: (i, 0)),
                   pl.BlockSpec((bs, max_army), lambda i: (i, 0))],
        out_shape=(
            jax.ShapeDtypeStruct((n_act // 128, 128), jnp.float32),
            jax.ShapeDtypeStruct((n_act, max_army), jnp.float32),
        ),
    )(pre, stats,
      p['es_gamma'].reshape(1, -1), p['es_beta'].reshape(1, -1),
      p['es_W2'].T, p['es_b2'].reshape(1, -1),
      p['as_gamma'].reshape(1, -1), p['as_beta'].reshape(1, -1),
      p['as_W2'].T, p['as_b2'].reshape(1, -1))

    return place.reshape(n), attack2d.reshape(n_act), army2d


# speedup vs baseline: 9.6753x; 1.0069x over previous
import functools

import jax
import jax.numpy as jnp
from jax import lax
from jax.experimental import pallas as pl
from jax.experimental.pallas import tpu as pltpu
from jax.experimental.pallas import tpu_sc as plsc

NC = 2
NS = 16
NW = NC * NS
C = 80


def _fill2d(ref, nrows, ncols, value):
    nb = ncols // 16
    v16 = jnp.full((16,), value, jnp.float32)

    def row(i, carry):
        for k in range(nb):
            ref[i, pl.ds(k * 16, 16)] = v16
        return carry

    lax.fori_loop(0, nrows, row, 0)


def _make_counts_sc(n_nodes, n_edges):
    ew = n_edges // NW
    nch = ew // C
    n_pad = ((n_nodes + 8 * NS - 1) // (8 * NS)) * (8 * NS)
    rps = n_pad // NS
    mesh = plsc.VectorSubcoreMesh(
        core_axis_name="c", subcore_axis_name="s",
        num_cores=NC, num_subcores=NS)
    out_type = jax.ShapeDtypeStruct((NC, n_pad, 16), jnp.float32)
    scratch = [
        pltpu.VMEM((nch, C), jnp.int32),
        pltpu.VMEM((C, 16), jnp.float32),
        pltpu.VMEM((rps, 16), jnp.float32),
        pltpu.VMEM_SHARED((n_pad, 16), jnp.float32),
        pltpu.SemaphoreType.DMA,
    ]

    def body(dstR, cnt_out, idx_d, ones, zcnt, cnt_sh, csem):
        c = lax.axis_index("c")
        s = lax.axis_index("s")
        wid = s * NC + c
        _fill2d(ones, C, 16, 1.0)
        _fill2d(zcnt, rps, 16, 0.0)
        pltpu.sync_copy(zcnt, cnt_sh.at[pl.ds(s * rps, rps)])
        plsc.subcore_barrier()
        pltpu.sync_copy(dstR.at[wid], idx_d)

        def fire(j, carry):
            pltpu.async_copy(ones, cnt_sh.at[idx_d.at[j]], csem, add=True)
            return carry

        lax.fori_loop(0, nch, fire, 0)

        def drain(j, carry):
            pltpu.make_async_copy(ones, cnt_sh.at[idx_d.at[0]], csem).wait()
            return carry

        lax.fori_loop(0, nch, drain, 0)
        plsc.subcore_barrier()
        pltpu.sync_copy(cnt_sh.at[pl.ds(s * rps, rps)], zcnt)
        pltpu.sync_copy(zcnt, cnt_out.at[c].at[pl.ds(s * rps, rps)])

    return pl.kernel(body, out_type=out_type, mesh=mesh,
                     scratch_types=scratch,
                     compiler_params=pltpu.CompilerParams(
                         use_tc_tiling_on_sc=False))


def _make_sage_sc(n_nodes, n_edges, d):
    ew = n_edges // NW
    nch = ew // C
    n_pad = ((n_nodes + 8 * NS - 1) // (8 * NS)) * (8 * NS)
    rps = n_pad // NS
    mesh = plsc.VectorSubcoreMesh(
        core_axis_name="c", subcore_axis_name="s",
        num_cores=NC, num_subcores=NS)

    NB = 4
    out_type = jax.ShapeDtypeStruct((NC, n_pad, d), jnp.float32)
    scratch = [
        pltpu.VMEM((nch, C), jnp.int32),
        pltpu.VMEM((nch, C), jnp.int32),
    ] + [pltpu.VMEM((C, d), jnp.float32) for _ in range(NB)] + [
        pltpu.VMEM((rps, d), jnp.float32),
        pltpu.VMEM_SHARED((n_pad, d), jnp.float32),
    ] + [pltpu.SemaphoreType.DMA for _ in range(2 * NB)]

    def body(srcR, dstR, h_hbm, agg_out, idx_s, idx_d, *rest):
        bufs = rest[:NB]
        zrows = rest[NB]
        agg_sh = rest[NB + 1]
        gsem = rest[NB + 2:2 * NB + 2]
        ssem = rest[2 * NB + 2:]
        c = lax.axis_index("c")
        s = lax.axis_index("s")
        wid = s * NC + c

        _fill2d(zrows, rps, d, 0.0)
        pltpu.sync_copy(zrows, agg_sh.at[pl.ds(s * rps, rps)])
        plsc.subcore_barrier()

        pltpu.sync_copy(srcR.at[wid], idx_s)
        pltpu.sync_copy(dstR.at[wid], idx_d)

        for k in range(NB):
            pltpu.async_copy(h_hbm.at[idx_s.at[k]], bufs[k], gsem[k])

        def it(jj, carry):
            for pp in range(NB):
                j = NB * jj + pp
                pltpu.make_async_copy(h_hbm.at[idx_s.at[0]], bufs[pp],
                                      gsem[pp]).wait()
                pltpu.async_copy(bufs[pp], agg_sh.at[idx_d.at[j]], ssem[pp],
                                 add=True)
                pltpu.make_async_copy(bufs[pp], agg_sh.at[idx_d.at[0]],
                                      ssem[pp]).wait()

                @pl.when(j + NB < nch)
                def _():
                    pltpu.async_copy(h_hbm.at[idx_s.at[j + NB]], bufs[pp],
                                     gsem[pp])
            return carry

        lax.fori_loop(0, nch // NB, it, 0)
        for pp in range(nch % NB):
            j = (nch // NB) * NB + pp
            pltpu.make_async_copy(h_hbm.at[idx_s.at[0]], bufs[pp],
                                  gsem[pp]).wait()
            pltpu.async_copy(bufs[pp], agg_sh.at[idx_d.at[j]], ssem[pp],
                             add=True)
            pltpu.make_async_copy(bufs[pp], agg_sh.at[idx_d.at[0]],
                                  ssem[pp]).wait()
        plsc.subcore_barrier()

        pltpu.sync_copy(agg_sh.at[pl.ds(s * rps, rps)], zrows)
        pltpu.sync_copy(zrows, agg_out.at[c].at[pl.ds(s * rps, rps)])

    return pl.kernel(body, out_type=out_type, mesh=mesh,
                     scratch_types=scratch,
                     compiler_params=pltpu.CompilerParams(
                         use_tc_tiling_on_sc=False))


def _make_edge_gather_sc(n_act, dproj):
    ew = n_act // NW
    nch = ew // C
    nv = dproj // 16
    NB = 3
    mesh = plsc.VectorSubcoreMesh(
        core_axis_name="c", subcore_axis_name="s",
        num_cores=NC, num_subcores=NS)
    out_type = (
        jax.ShapeDtypeStruct((n_act, dproj), jnp.float32),
        jax.ShapeDtypeStruct((NW, 2, dproj), jnp.float32),
    )
    scratch = (
        [pltpu.VMEM((nch, C), jnp.int32), pltpu.VMEM((nch, C), jnp.int32)]
        + [pltpu.VMEM((C, dproj), jnp.float32) for _ in range(3 * NB)]
        + [pltpu.VMEM((2, dproj), jnp.float32)]
        + [pltpu.SemaphoreType.DMA for _ in range(3 * NB)]
    )

    def body(srcR, tgtR, gas_hbm, gat_hbm, pre, stats_out,
             idx_s, idx_t, *rest):
        abuf = rest[0:NB]
        bbuf = rest[NB:2 * NB]
        obuf = rest[2 * NB:3 * NB]
        st = rest[3 * NB]
        gasem = rest[3 * NB + 1:4 * NB + 1]
        gbsem = rest[4 * NB + 1:5 * NB + 1]
        wsem = rest[5 * NB + 1:6 * NB + 1]
        c = lax.axis_index("c")
        s = lax.axis_index("s")
        wid = s * NC + c
        pltpu.sync_copy(srcR.at[wid], idx_s)
        pltpu.sync_copy(tgtR.at[wid], idx_t)
        base = wid * ew

        for k in range(NB):
            pltpu.async_copy(gas_hbm.at[idx_s.at[k]], abuf[k], gasem[k])
            pltpu.async_copy(gat_hbm.at[idx_t.at[k]], bbuf[k], gbsem[k])

        zero = jnp.zeros((16,), jnp.float32)
        stats0 = tuple(zero for _ in range(2 * nv))

        def valu(pp, stats):
            a, b, o = abuf[pp], bbuf[pp], obuf[pp]

            def row(i, stats):
                stats = list(stats)
                for k in range(nv):
                    v = a[i, pl.ds(k * 16, 16)] + b[i, pl.ds(k * 16, 16)]
                    o[i, pl.ds(k * 16, 16)] = v
                    stats[k] = stats[k] + v
                    stats[nv + k] = stats[nv + k] + v * v
                return tuple(stats)

            return lax.fori_loop(0, C, row, stats)

        def step(pp, j, guard_w, stats):
            pltpu.make_async_copy(gas_hbm.at[idx_s.at[0]], abuf[pp],
                                  gasem[pp]).wait()
            pltpu.make_async_copy(gat_hbm.at[idx_t.at[0]], bbuf[pp],
                                  gbsem[pp]).wait()

            @pl.when(guard_w)
            def _():
                pltpu.make_async_copy(
                    obuf[pp], pre.at[pl.ds(base, C)], wsem[pp]).wait()

            stats = valu(pp, stats)
            pltpu.async_copy(obuf[pp], pre.at[pl.ds(base + j * C, C)],
                             wsem[pp])

            @pl.when(j + NB < nch)
            def _():
                pltpu.async_copy(gas_hbm.at[idx_s.at[j + NB]], abuf[pp],
                                 gasem[pp])
                pltpu.async_copy(gat_hbm.at[idx_t.at[j + NB]], bbuf[pp],
                                 gbsem[pp])
            return stats

        def it(jj, stats):
            for pp in range(NB):
                stats = step(pp, NB * jj + pp, jj >= 1, stats)
            return stats

        stats = lax.fori_loop(0, nch // NB, it, stats0)
        for pp in range(nch % NB):
            j = (nch // NB) * NB + pp
            stats = step(pp, j, j >= NB, stats)
        for pp in range(NB):
            pltpu.make_async_copy(obuf[pp], pre.at[pl.ds(base, C)],
                                  wsem[pp]).wait()
        for k in range(nv):
            st[0, pl.ds(k * 16, 16)] = stats[k]
            st[1, pl.ds(k * 16, 16)] = stats[nv + k]
        pltpu.sync_copy(st, stats_out.at[wid])

    return pl.kernel(body, out_type=out_type, mesh=mesh,
                     scratch_types=scratch,
                     compiler_params=pltpu.CompilerParams(
                         use_tc_tiling_on_sc=False))


def _proj_body(h_ref, wT_ref, out_ref):
    out_ref[...] = jnp.dot(h_ref[...], wT_ref[...],
                           preferred_element_type=jnp.float32)


def _sage_core(n, agg_ref, cnt_ref, h_ref, bl_ref, wrT_ref, g_ref, b_ref):
    a = agg_ref[0][:n] + agg_ref[1][:n]
    cnt = cnt_ref[0][:n, 0:1] + cnt_ref[1][:n, 0:1]
    out = (a / jnp.maximum(cnt, 1.0)
           + jnp.dot(h_ref[...], wrT_ref[...], preferred_element_type=jnp.float32)
           + bl_ref[...])
    nrm = jnp.sqrt(jnp.sum(out * out, axis=1, keepdims=True))
    out = out / jnp.maximum(nrm, 1e-12)
    mu = jnp.mean(out, axis=0, keepdims=True)
    var = jnp.mean((out - mu) ** 2, axis=0, keepdims=True)
    out = (out - mu) / jnp.sqrt(var + 1e-5) * g_ref[...] + b_ref[...]
    return jnp.maximum(out, 0.0)


def _sage_dense_body(n, agg_ref, cnt_ref, h_ref, bl_ref, wrT_ref,
                     g_ref, b_ref, wlnT_ref, out_ref, outp_ref):
    h = _sage_core(n, agg_ref, cnt_ref, h_ref, bl_ref, wrT_ref, g_ref, b_ref)
    out_ref[...] = h
    outp_ref[...] = jnp.dot(h, wlnT_ref[...], preferred_element_type=jnp.float32)


def _final_dense_body(n, agg_ref, cnt_ref, h_ref, bl_ref, wrT_ref,
                      g_ref, b_ref, plw1T_ref, plb1_ref, plg_ref, plb_ref,
                      plw2T_ref, plb2_ref, wsT_ref, wtT_ref,
                      place_ref, gas_ref, gat_ref):
    h3 = _sage_core(n, agg_ref, cnt_ref, h_ref, bl_ref, wrT_ref, g_ref, b_ref)
    hc5 = jnp.clip(h3, -5.0, 5.0)
    t = jnp.dot(hc5, plw1T_ref[...], preferred_element_type=jnp.float32) + plb1_ref[...]
    tmu = jnp.mean(t, axis=0, keepdims=True)
    tvar = jnp.mean((t - tmu) ** 2, axis=0, keepdims=True)
    t = jnp.maximum((t - tmu) / jnp.sqrt(tvar + 1e-5) * plg_ref[...] + plb_ref[...],
                    0.0)
    place = jnp.dot(t, plw2T_ref[...], preferred_element_type=jnp.float32) + plb2_ref[...]
    place_ref[...] = jnp.clip(place, -15.0, 15.0)
    hc3 = jnp.clip(h3, -3.0, 3.0)
    gas_ref[...] = jnp.dot(hc3, wsT_ref[...], preferred_element_type=jnp.float32)
    gat_ref[...] = jnp.dot(hc3, wtT_ref[...], preferred_element_type=jnp.float32)


def _finalize_body(n_act, pre_ref, stats_ref, esg_ref, esb_ref,
                   esw2T_ref, esb2_ref, asg_ref, asb_ref, asw2T_ref,
                   asb2_ref, attack_ref, army_ref):
    x = pre_ref[...]
    mu = jnp.sum(stats_ref[:, 0, :], axis=0, keepdims=True) * (1.0 / n_act)
    s2 = jnp.sum(stats_ref[:, 1, :], axis=0, keepdims=True)
    var = s2 * (1.0 / n_act) - mu * mu
    inv = 1.0 / jnp.sqrt(var + 1e-5)
    xn = (x - mu) * inv
    e1 = jnp.maximum(xn[:, :64] * esg_ref[...] + esb_ref[...], 0.0)
    a1 = jnp.maximum(xn[:, 64:] * asg_ref[...] + asb_ref[...], 0.0)
    attack = jnp.dot(e1, esw2T_ref[...], preferred_element_type=jnp.float32) + esb2_ref[...]
    attack_ref[...] = jnp.clip(attack, -15.0, 15.0).reshape(attack_ref.shape)
    army = jnp.dot(a1, asw2T_ref[...], preferred_element_type=jnp.float32) + asb2_ref[...]
    army_ref[...] = jnp.clip(army, -15.0, 15.0)


def _full(shape):
    return pl.BlockSpec(shape, lambda *_: tuple(0 for _ in shape))


def kernel(x, edge_index, action_edges, army_counts, params):
    n, dfeat = x.shape
    n_edges = edge_index.shape[1]
    n_act = action_edges.shape[0]
    embed = params['sage0_Wl'].shape[0]
    p = params

    src = edge_index[0].astype(jnp.int32).reshape(NW, n_edges // (NW * C), C)
    dst = edge_index[1].astype(jnp.int32).reshape(NW, n_edges // (NW * C), C)
    asrc = action_edges[:, 0].astype(jnp.int32).reshape(NW, n_act // (NW * C), C)
    atgt = action_edges[:, 1].astype(jnp.int32).reshape(NW, n_act // (NW * C), C)

    counts_sc = _make_counts_sc(n, n_edges)
    sage_sc = _make_sage_sc(n, n_edges, embed)

    xp = pl.pallas_call(
        _proj_body,
        out_shape=jax.ShapeDtypeStruct((n, embed), jnp.float32),
    )(x, p['sage0_Wl'].T)

    cnt = counts_sc(dst)
    agg0 = sage_sc(src, dst, xp)

    def dense(aggv, h, i):
        wrT = p['sage%d_Wr' % i].T
        bl = p['sage%d_bl' % i].reshape(1, -1)
        g = p['sage%d_gamma' % i].reshape(1, -1)
        b = p['sage%d_beta' % i].reshape(1, -1)
        wlnT = p['sage%d_Wl' % (i + 1)].T
        return pl.pallas_call(
            functools.partial(_sage_dense_body, n),
            out_shape=(jax.ShapeDtypeStruct((n, embed), jnp.float32),
                       jax.ShapeDtypeStruct((n, embed), jnp.float32)),
        )(aggv, cnt, h, bl, wrT, g, b, wlnT)

    h1, h1p = dense(agg0, x, 0)
    agg1 = sage_sc(src, dst, h1p)
    h2, h2p = dense(agg1, h1, 1)
    agg2 = sage_sc(src, dst, h2p)

    wsT = jnp.concatenate([p['es_W1'][:, :embed].T, p['as_W1'][:, :embed].T],
                          axis=1)
    wtT = jnp.concatenate([p['es_W1'][:, embed:].T, p['as_W1'][:, embed:].T],
                          axis=1)
    place, gas, gat = pl.pallas_call(
        functools.partial(_final_dense_body, n),
        out_shape=(
            jax.ShapeDtypeStruct((n, 1), jnp.float32),
            jax.ShapeDtypeStruct((n, 2 * embed), jnp.float32),
            jax.ShapeDtypeStruct((n, 2 * embed), jnp.float32),
        ),
    )(agg2, cnt, h2, p['sage2_bl'].reshape(1, -1),
      p['sage2_Wr'].T, p['sage2_gamma'].reshape(1, -1),
      p['sage2_beta'].reshape(1, -1),
      p['pl_W1'].T, p['pl_b1'].reshape(1, -1),
      p['pl_gamma'].reshape(1, -1), p['pl_beta'].reshape(1, -1),
      p['pl_W2'].T, p['pl_b2'].reshape(1, -1), wsT, wtT)

    edge_sc = _make_edge_gather_sc(n_act, 2 * embed)
    pre, stats = edge_sc(asrc, atgt, gas, gat)

    bs = 16384
    grid = ((n_act + bs - 1) // bs,)
    max_army = p['as_W2'].shape[0]
    arows = bs * max_army // 128
    attack2d, army2d = pl.pallas_call(
        functools.partial(_finalize_body, n_act),
        grid=grid,
        in_specs=[pl.BlockSpec((bs, 2 * embed), lambda i: (i, 0)),
                  _full((NW, 2, 2 * embed)),
                  _full((1, embed)), _full((1, embed)),
                  _full((embed, 1)), _full((1, 1)),
                  _full((1, embed)), _full((1, embed)),
                  _full((embed, max_army)), _full((1, max_army))],
        out_specs=[pl.BlockSpec((bs // 128, 128), lambda i: (i, 0)),
                   pl.BlockSpec((bs, max_army), lambda i: (i, 0))],
        out_shape=(
            jax.ShapeDtypeStruct((n_act // 128, 128), jnp.float32),
            jax.ShapeDtypeStruct((n_act, max_army), jnp.float32),
        ),
    )(pre, stats,
      p['es_gamma'].reshape(1, -1), p['es_beta'].reshape(1, -1),
      p['es_W2'].T, p['es_b2'].reshape(1, -1),
      p['as_gamma'].reshape(1, -1), p['as_beta'].reshape(1, -1),
      p['as_W2'].T, p['as_b2'].reshape(1, -1))

    return place.reshape(n), attack2d.reshape(n_act), army2d
